# Initial kernel scaffold; baseline (speedup 1.0000x reference)
import jax
import jax.numpy as jnp
from jax.experimental import pallas as pl


def kernel(x, tables):
    raise NotImplementedError("write your pallas kernel here")


# speedup vs baseline: 1.1578x; 1.1578x over previous
import jax
import jax.numpy as jnp
from jax import lax
from jax.experimental import pallas as pl
from jax.experimental.pallas import tpu as pltpu
from jax.experimental.pallas import tpu_sc as plsc

_BATCH = 16384
_N_FIELDS = 26
_VOCAB = 100000
_EMB = 16

_NC = 2
_NS = 16
_NW = _NC * _NS
_L = 16

_TOTAL = _BATCH * _N_FIELDS
_PER_W = _TOTAL // _NW
_IDX_PER_DMA = 128
_CHUNK = 1664
_DMAS_PER_CHUNK = _CHUNK // _IDX_PER_DMA
_N_CHUNKS = _PER_W // _CHUNK
_PAT = 208
_PAT_VECS = _PAT // _L


def _body(tab_hbm, xf_hbm, out_hbm, idx_v, b0, b1, b2,
          g0, g1, g2, o0, o1, o2):
    bufs = (b0, b1, b2)
    gsems = (g0, g1, g2)
    osems = (o0, o1, o2)

    wid = lax.axis_index("s") * _NC + lax.axis_index("c")
    base = wid * _PER_W

    pltpu.sync_copy(xf_hbm.at[pl.ds(base, _PER_W)], idx_v)

    iota = lax.iota(jnp.int32, _L)
    pats = [((iota + _L * j) % _N_FIELDS) * _VOCAB for j in range(_PAT_VECS)]

    @pl.loop(0, _PER_W // _PAT)
    def _add(g):
        off0 = g * _PAT
        for j in range(_PAT_VECS):
            sl = pl.ds(off0 + j * _L, _L)
            idx_v[sl] = idx_v[sl] + pats[j]

    gds = [None] * _N_CHUNKS
    ods = [None] * _N_CHUNKS

    def fire_gathers(c):
        b = c % 3
        ds = []
        for j in range(_DMAS_PER_CHUNK):
            isl = idx_v.at[pl.ds((c * _DMAS_PER_CHUNK + j) * _IDX_PER_DMA,
                                 _IDX_PER_DMA)]
            dst = bufs[b].at[pl.ds(j * _IDX_PER_DMA, _IDX_PER_DMA), :]
            ds.append(pltpu.async_copy(tab_hbm.at[isl], dst, gsems[b]))
        gds[c] = ds

    def drain_and_out(c):
        b = c % 3
        for d in gds[c]:
            d.wait()
        ods[c] = pltpu.async_copy(
            bufs[b], out_hbm.at[pl.ds(base + c * _CHUNK, _CHUNK), :], osems[b])

    fire_gathers(0)
    for c in range(1, _N_CHUNKS + 1):
        if c < _N_CHUNKS:
            if c >= 3:
                ods[c - 3].wait()
            fire_gathers(c)
        drain_and_out(c - 1)
    for c in range(_N_CHUNKS - 3, _N_CHUNKS):
        ods[c].wait()


_gather_call = pl.kernel(
    _body,
    out_type=jax.ShapeDtypeStruct((_TOTAL, _EMB), jnp.float32),
    mesh=plsc.VectorSubcoreMesh(core_axis_name="c", subcore_axis_name="s",
                                num_cores=_NC, num_subcores=_NS),
    scratch_types=(
        [pltpu.VMEM((_PER_W,), jnp.int32)]
        + [pltpu.VMEM((_CHUNK, _EMB), jnp.float32) for _ in range(3)]
        + [pltpu.SemaphoreType.DMA for _ in range(6)]
    ),
    compiler_params=pltpu.CompilerParams(use_tc_tiling_on_sc=False),
)


def kernel(x, tables):
    xf = x.astype(jnp.int32).reshape(_TOTAL)
    tab = tables.reshape(_N_FIELDS * _VOCAB, _EMB)
    out = _gather_call(tab, xf)
    return out.reshape(_BATCH, _N_FIELDS * _EMB)
